# baseline (speedup 1.0000x reference)
import functools

import jax
import jax.numpy as jnp
from jax.experimental import pallas as pl

_ROWS_PER_BLOCK = 128
_SUB = 8
_LANES = 128
_ROW_GROUP = 1


def _asnc_ln_body(t8_ref, y8_ref, t2_ref, y2_ref, gamma_ref, beta_ref,
                  x_ref, o_ref, *, n_thresh, h):
    Rg = x_ref.shape[0]
    n_chunks = h // _LANES

    def tj(j, sl):
        if True:
            return t2_ref[j:j + 1, sl]
        return t8_ref[j:j + 1, :, sl]

    def yj(j, sl):
        if True:
            return y2_ref[j:j + 1, sl]
        return y8_ref[j:j + 1, :, sl]

    for r in range(0, Rg, _ROW_GROUP):
        rs = slice(r, r + _ROW_GROUP)
        s = jnp.zeros((_ROW_GROUP, _SUB, _LANES), jnp.float32)
        s2 = jnp.zeros((_ROW_GROUP, _SUB, _LANES), jnp.float32)
        for c in range(n_chunks):
            sl = slice(c * _LANES, (c + 1) * _LANES)
            xc = x_ref[rs, :, sl]
            v = jnp.where(xc > tj(0, sl), yj(1, sl), yj(0, sl))
            for j in range(1, n_thresh):
                v = jnp.where(xc > tj(j, sl), yj(j + 1, sl), v)
            o_ref[rs, :, sl] = v
            s = s + v
            s2 = s2 + v * v
        m = jnp.sum(s, axis=-1, keepdims=True) * (1.0 / h)
        ex2 = jnp.sum(s2, axis=-1, keepdims=True) * (1.0 / h)
        var = ex2 - m * m
        inv = jax.lax.rsqrt(var + jnp.float32(1e-5))
        for c in range(n_chunks):
            sl = slice(c * _LANES, (c + 1) * _LANES)
            v = o_ref[rs, :, sl]
            o_ref[rs, :, sl] = ((v - m) * inv * gamma_ref[0:1, :, sl]
                                + beta_ref[0:1, :, sl])


@jax.jit
def kernel(x, thresholds, y, gamma, beta):
    shape = x.shape
    H = shape[-1]
    Km1 = thresholds.shape[1]
    K = y.shape[1]
    x3 = x.reshape(-1, _SUB, H)
    G = x3.shape[0]

    t2 = thresholds.T
    y2 = y.T
    t8 = jnp.broadcast_to(t2[:, None, :], (Km1, _SUB, H))
    y8 = jnp.broadcast_to(y2[:, None, :], (K, _SUB, H))
    gamma3 = jnp.broadcast_to(gamma[None, None, :], (1, _SUB, H))
    beta3 = jnp.broadcast_to(beta[None, None, :], (1, _SUB, H))

    Rg = _ROWS_PER_BLOCK // _SUB
    grid = (G // Rg,)

    out = pl.pallas_call(
        functools.partial(_asnc_ln_body, n_thresh=Km1, h=H),
        grid=grid,
        in_specs=[
            pl.BlockSpec((Km1, _SUB, H), lambda i: (0, 0, 0)),
            pl.BlockSpec((K, _SUB, H), lambda i: (0, 0, 0)),
            pl.BlockSpec((Km1, H), lambda i: (0, 0)),
            pl.BlockSpec((K, H), lambda i: (0, 0)),
            pl.BlockSpec((1, _SUB, H), lambda i: (0, 0, 0)),
            pl.BlockSpec((1, _SUB, H), lambda i: (0, 0, 0)),
            pl.BlockSpec((Rg, _SUB, H), lambda i: (i, 0, 0)),
        ],
        out_specs=pl.BlockSpec((Rg, _SUB, H), lambda i: (i, 0, 0)),
        out_shape=jax.ShapeDtypeStruct((G, _SUB, H), x.dtype),
    )(t8, y8, t2, y2, gamma3, beta3, x3)
    return out.reshape(shape)


# speedup vs baseline: 1.0044x; 1.0044x over previous
import functools

import jax
import jax.numpy as jnp
from jax.experimental import pallas as pl

_ROWS_PER_BLOCK = 128
_SUB = 8
_LANES = 128
_ROW_GROUP = 1


def _asnc_ln_body(t8_ref, y8_ref, t2_ref, y2_ref, gamma_ref, beta_ref,
                  x_ref, o_ref, *, n_thresh, h):
    Rg = x_ref.shape[0]
    n_chunks = h // _LANES

    def tj(j, sl):
        if j % 3 != 0:
            return t2_ref[j:j + 1, sl]
        return t8_ref[j:j + 1, :, sl]

    def yj(j, sl):
        if j % 3 != 0:
            return y2_ref[j:j + 1, sl]
        return y8_ref[j:j + 1, :, sl]

    for r in range(0, Rg, _ROW_GROUP):
        rs = slice(r, r + _ROW_GROUP)
        s = jnp.zeros((_ROW_GROUP, _SUB, _LANES), jnp.float32)
        s2 = jnp.zeros((_ROW_GROUP, _SUB, _LANES), jnp.float32)
        for c in range(n_chunks):
            sl = slice(c * _LANES, (c + 1) * _LANES)
            xc = x_ref[rs, :, sl]
            v = jnp.where(xc > tj(0, sl), yj(1, sl), yj(0, sl))
            for j in range(1, n_thresh):
                v = jnp.where(xc > tj(j, sl), yj(j + 1, sl), v)
            o_ref[rs, :, sl] = v
            s = s + v
            s2 = s2 + v * v
        m = jnp.sum(s, axis=-1, keepdims=True) * (1.0 / h)
        ex2 = jnp.sum(s2, axis=-1, keepdims=True) * (1.0 / h)
        var = ex2 - m * m
        inv = jax.lax.rsqrt(var + jnp.float32(1e-5))
        for c in range(n_chunks):
            sl = slice(c * _LANES, (c + 1) * _LANES)
            v = o_ref[rs, :, sl]
            o_ref[rs, :, sl] = ((v - m) * inv * gamma_ref[0:1, :, sl]
                                + beta_ref[0:1, :, sl])


@jax.jit
def kernel(x, thresholds, y, gamma, beta):
    shape = x.shape
    H = shape[-1]
    Km1 = thresholds.shape[1]
    K = y.shape[1]
    x3 = x.reshape(-1, _SUB, H)
    G = x3.shape[0]

    t2 = thresholds.T
    y2 = y.T
    t8 = jnp.broadcast_to(t2[:, None, :], (Km1, _SUB, H))
    y8 = jnp.broadcast_to(y2[:, None, :], (K, _SUB, H))
    gamma3 = jnp.broadcast_to(gamma[None, None, :], (1, _SUB, H))
    beta3 = jnp.broadcast_to(beta[None, None, :], (1, _SUB, H))

    Rg = _ROWS_PER_BLOCK // _SUB
    grid = (G // Rg,)

    out = pl.pallas_call(
        functools.partial(_asnc_ln_body, n_thresh=Km1, h=H),
        grid=grid,
        in_specs=[
            pl.BlockSpec((Km1, _SUB, H), lambda i: (0, 0, 0)),
            pl.BlockSpec((K, _SUB, H), lambda i: (0, 0, 0)),
            pl.BlockSpec((Km1, H), lambda i: (0, 0)),
            pl.BlockSpec((K, H), lambda i: (0, 0)),
            pl.BlockSpec((1, _SUB, H), lambda i: (0, 0, 0)),
            pl.BlockSpec((1, _SUB, H), lambda i: (0, 0, 0)),
            pl.BlockSpec((Rg, _SUB, H), lambda i: (i, 0, 0)),
        ],
        out_specs=pl.BlockSpec((Rg, _SUB, H), lambda i: (i, 0, 0)),
        out_shape=jax.ShapeDtypeStruct((G, _SUB, H), x.dtype),
    )(t8, y8, t2, y2, gamma3, beta3, x3)
    return out.reshape(shape)
